# baseline (speedup 1.0000x reference)
import jax
import jax.numpy as jnp
from jax import lax
from jax.experimental import pallas as pl
from jax.experimental.pallas import tpu as pltpu
from jax.experimental.pallas import tpu_sc as plsc

f32 = jnp.float32
i32 = jnp.int32

N_NODES = 10000
D = 128
NG = 64
NCLS = 18
NSC = 2
NT = 16
NW = NSC * NT
FPT = D // NW
N_PAD = 10240
EC = 4096
E_CHUNKS = 80
E_PAD = EC * E_CHUNKS
DEG_CHUNKS = E_PAD // NW // 128
ROW_BLK = 512
N_BLKS = N_PAD // ROW_BLK

_SC_MESH = plsc.VectorSubcoreMesh(core_axis_name="c", subcore_axis_name="s")
_SC_PARAMS = pltpu.CompilerParams(needs_layout_passes=False)


def _spmm_body(gt_hbm, src_hbm, dst_hbm, w_hbm, out,
               slab, accv, src0, src1, dst0, dst1, w0, w1, sem0, sem1):
    c = lax.axis_index("c")
    s = lax.axis_index("s")
    wid = c * NT + s
    srcb = (src0, src1)
    dstb = (dst0, dst1)
    wb = (w0, w1)
    sem = (sem0, sem1)

    pltpu.sync_copy(gt_hbm.at[pl.ds(wid * FPT, FPT)], slab)

    def az(i, carry):
        for f in range(FPT):
            accv[f, pl.ds(i * 16, 16)] = jnp.zeros((16,), f32)
        return carry
    lax.fori_loop(0, N_PAD // 16, az, 0)

    def fire(ch, b):
        es = pl.ds(ch * EC, EC)
        pltpu.async_copy(src_hbm.at[es], srcb[b], sem[b])
        pltpu.async_copy(dst_hbm.at[es], dstb[b], sem[b])
        pltpu.async_copy(w_hbm.at[es], wb[b], sem[b])

    def drain(ch, b):
        es = pl.ds(ch * EC, EC)
        pltpu.make_async_copy(src_hbm.at[es], srcb[b], sem[b]).wait()
        pltpu.make_async_copy(dst_hbm.at[es], dstb[b], sem[b]).wait()
        pltpu.make_async_copy(w_hbm.at[es], wb[b], sem[b]).wait()

    fire(0, 0)
    fire(1, 1)

    def pair(p, carry):
        for b in range(2):
            ch = p * 2 + b
            drain(ch, b)

            @plsc.parallel_loop(0, EC // 16, unroll=8)
            def grp(i):
                sl = pl.ds(i * 16, 16)
                s16 = srcb[b][sl]
                d16 = dstb[b][sl]
                w16 = wb[b][sl]
                for f in range(FPT):
                    ff = jnp.full((16,), f, i32)
                    vals = plsc.load_gather(slab, [ff, s16])
                    plsc.addupdate_scatter(accv, [ff, d16], vals * w16)
            nxt = ch + 2

            @pl.when(nxt < E_CHUNKS)
            def _():
                fire(nxt, b)
        return carry
    lax.fori_loop(0, E_CHUNKS // 2, pair, 0)

    pltpu.sync_copy(accv, out.at[pl.ds(wid * FPT, FPT)])


def _spmm(gt, srcf, dstf, wf):
    return pl.kernel(
        _spmm_body,
        out_type=jax.ShapeDtypeStruct((D, N_PAD), f32),
        mesh=_SC_MESH,
        compiler_params=_SC_PARAMS,
        scratch_types=[
            pltpu.VMEM((FPT, N_PAD), f32),
            pltpu.VMEM((FPT, N_PAD), f32),
            pltpu.VMEM((EC,), i32),
            pltpu.VMEM((EC,), i32),
            pltpu.VMEM((EC,), i32),
            pltpu.VMEM((EC,), i32),
            pltpu.VMEM((EC,), f32),
            pltpu.VMEM((EC,), f32),
            pltpu.SemaphoreType.DMA,
            pltpu.SemaphoreType.DMA,
        ],
    )(gt, srcf, dstf, wf)


def _deg_body(dst_hbm, w_hbm, out, degp, dst_v, w_v):
    c = lax.axis_index("c")
    s = lax.axis_index("s")
    wid = c * NT + s

    def dz(i, carry):
        degp[pl.ds(i * 16, 16)] = jnp.zeros((16,), f32)
        return carry
    lax.fori_loop(0, N_PAD // 16, dz, 0)

    pltpu.sync_copy(dst_hbm.at[wid], dst_v)
    pltpu.sync_copy(w_hbm.at[wid], w_v)

    @plsc.parallel_loop(0, DEG_CHUNKS, unroll=2)
    def eb(r):
        for gi in range(8):
            sl = pl.ds(gi * 16, 16)
            plsc.addupdate_scatter(degp, [dst_v[r, sl]], w_v[r, sl])

    pltpu.sync_copy(degp, out.at[wid])


def _deg(dst3, w3):
    return pl.kernel(
        _deg_body,
        out_type=jax.ShapeDtypeStruct((NW, N_PAD), f32),
        mesh=_SC_MESH,
        compiler_params=_SC_PARAMS,
        scratch_types=[
            pltpu.VMEM((N_PAD,), f32),
            pltpu.VMEM((DEG_CHUNKS, 128), i32),
            pltpu.VMEM((DEG_CHUNKS, 128), f32),
        ],
    )(dst3, w3)


def _dred_body(degp_ref, out_ref):
    out_ref[...] = lax.rsqrt(1.0 + jnp.sum(degp_ref[...], axis=0,
                                           keepdims=True))


def _dred(degp):
    return pl.pallas_call(
        _dred_body,
        grid=(N_BLKS,),
        in_specs=[pl.BlockSpec((NW, ROW_BLK), lambda i: (0, i))],
        out_specs=pl.BlockSpec((1, ROW_BLK), lambda i: (0, i)),
        out_shape=jax.ShapeDtypeStruct((1, N_PAD), f32),
    )(degp)


def _g0_body(x_ref, w0t_ref, dinv_ref, g0_ref):
    xw = lax.dot_general(w0t_ref[...], x_ref[...],
                         (((1,), (1,)), ((), ())),
                         preferred_element_type=f32)
    g0_ref[...] = dinv_ref[...] * xw


def _g0(x_pad, w0t, dinvT):
    return pl.pallas_call(
        _g0_body,
        grid=(N_BLKS,),
        in_specs=[
            pl.BlockSpec((ROW_BLK, D), lambda i: (i, 0)),
            pl.BlockSpec((D, D), lambda i: (0, 0)),
            pl.BlockSpec((1, ROW_BLK), lambda i: (0, i)),
        ],
        out_specs=pl.BlockSpec((D, ROW_BLK), lambda i: (0, i)),
        out_shape=jax.ShapeDtypeStruct((D, N_PAD), f32),
    )(x_pad, w0t, dinvT)


def _layer_body(p_ref, g_ref, dinv_ref, b_ref, wt_ref, out_ref):
    dv = dinv_ref[...]
    h = dv * (p_ref[...] + g_ref[...]) + b_ref[...]
    h = jnp.maximum(h, 0.0)
    out_ref[...] = dv * jnp.dot(wt_ref[...], h, preferred_element_type=f32)


def _layer(p, g, dinvT, bcol, wt):
    return pl.pallas_call(
        _layer_body,
        grid=(N_BLKS,),
        in_specs=[
            pl.BlockSpec((D, ROW_BLK), lambda i: (0, i)),
            pl.BlockSpec((D, ROW_BLK), lambda i: (0, i)),
            pl.BlockSpec((1, ROW_BLK), lambda i: (0, i)),
            pl.BlockSpec((D, 1), lambda i: (0, 0)),
            pl.BlockSpec((D, D), lambda i: (0, 0)),
        ],
        out_specs=pl.BlockSpec((D, ROW_BLK), lambda i: (0, i)),
        out_shape=jax.ShapeDtypeStruct((D, N_PAD), f32),
    )(p, g, dinvT, bcol, wt)


def _pool_body(p_ref, g_ref, dinv_ref, b_ref, batch_ref, out_ref):
    i = pl.program_id(0)
    h = dinv_ref[...] * (p_ref[...] + g_ref[...]) + b_ref[...]
    onehot = (batch_ref[...] ==
              lax.broadcasted_iota(i32, (ROW_BLK, NG), 1)).astype(f32)
    acc = jnp.dot(h, onehot, preferred_element_type=f32)

    @pl.when(i == 0)
    def _():
        out_ref[...] = acc

    @pl.when(i != 0)
    def _():
        out_ref[...] += acc


def _pool(p, g, dinvT, bcol, batch_col):
    return pl.pallas_call(
        _pool_body,
        grid=(N_BLKS,),
        in_specs=[
            pl.BlockSpec((D, ROW_BLK), lambda i: (0, i)),
            pl.BlockSpec((D, ROW_BLK), lambda i: (0, i)),
            pl.BlockSpec((1, ROW_BLK), lambda i: (0, i)),
            pl.BlockSpec((D, 1), lambda i: (0, 0)),
            pl.BlockSpec((ROW_BLK, 1), lambda i: (i, 0)),
        ],
        out_specs=pl.BlockSpec((D, NG), lambda i: (0, 0)),
        out_shape=jax.ShapeDtypeStruct((D, NG), f32),
    )(p, g, dinvT, bcol, batch_col)


def _head_body(pooled_ref, fc1wt_ref, fc1b_ref, fc2wt_ref, fc2b_ref,
               gam_ref, bet_ref, out_ref):
    p = pooled_ref[...]
    mean = jnp.mean(p, axis=1, keepdims=True)
    var = jnp.mean((p - mean) ** 2, axis=1, keepdims=True)
    hn = (p - mean) * lax.rsqrt(var + 1e-5) * gam_ref[...] + bet_ref[...]
    hf = jnp.dot(fc1wt_ref[...], hn, preferred_element_type=f32) + fc1b_ref[...]
    hf = jnp.maximum(hf, 0.0)
    logits = jnp.dot(fc2wt_ref[...], hf, preferred_element_type=f32) + fc2b_ref[...]
    row = lax.broadcasted_iota(i32, (D, NG), 0)
    lm = jnp.where(row < NCLS, logits, -1e30)
    mx = jnp.max(lm, axis=0, keepdims=True)
    lse = jnp.log(jnp.sum(jnp.exp(lm - mx), axis=0, keepdims=True)) + mx
    out_ref[...] = logits - lse


def _head(pooledT, fc1wt, fc1b_col, fc2wt_pad, fc2b_col, gam_col, bet_col):
    return pl.pallas_call(
        _head_body,
        out_shape=jax.ShapeDtypeStruct((D, NG), f32),
    )(pooledT, fc1wt, fc1b_col, fc2wt_pad, fc2b_col, gam_col, bet_col)


def kernel(x, edge_index, edge_type, batch, W, b,
           fc1_W, fc1_b, fc2_W, fc2_b, bn_gamma, bn_beta):
    n = x.shape[0]
    ne = edge_index.shape[1]
    n_conv = W.shape[0]

    src = edge_index[0].astype(i32)
    dst = edge_index[1].astype(i32)
    w = edge_type.astype(f32)

    epad = E_PAD - ne
    srcf = jnp.concatenate([src, jnp.zeros((epad,), i32)])
    dstf = jnp.concatenate([dst, jnp.zeros((epad,), i32)])
    wf = jnp.concatenate([w, jnp.zeros((epad,), f32)])
    dst3 = dstf.reshape(NW, DEG_CHUNKS, 128)
    w3 = wf.reshape(NW, DEG_CHUNKS, 128)

    x_pad = jnp.concatenate([x.astype(f32), jnp.zeros((N_PAD - n, D), f32)])
    batch_col = jnp.concatenate([batch.astype(i32),
                                 jnp.full((N_PAD - n,), NG, i32)]).reshape(N_PAD, 1)

    Wt = jnp.swapaxes(W, 1, 2).astype(f32)

    degp = _deg(dst3, w3)
    dinvT = _dred(degp)
    g = _g0(x_pad, Wt[0], dinvT)

    p = None
    for i in range(n_conv):
        p = _spmm(g, srcf, dstf, wf)
        if i + 1 < n_conv:
            g = _layer(p, g, dinvT, b[i].reshape(D, 1), Wt[i + 1])

    pooledT = _pool(p, g, dinvT, b[n_conv - 1].reshape(D, 1), batch_col)

    fc2wt_pad = jnp.zeros((D, D), f32).at[:NCLS, :].set(fc2_W.T.astype(f32))
    fc2b_col = jnp.zeros((D, 1), f32).at[:NCLS, 0].set(fc2_b.astype(f32))
    outT = _head(pooledT, fc1_W.T.astype(f32), fc1_b.reshape(D, 1).astype(f32),
                 fc2wt_pad, fc2b_col, bn_gamma.reshape(D, 1).astype(f32),
                 bn_beta.reshape(D, 1).astype(f32))
    return outT[:NCLS, :].T


# speedup vs baseline: 1.0168x; 1.0168x over previous
import jax
import jax.numpy as jnp
from jax import lax
from jax.experimental import pallas as pl
from jax.experimental.pallas import tpu as pltpu
from jax.experimental.pallas import tpu_sc as plsc

f32 = jnp.float32
i32 = jnp.int32

N_NODES = 10000
D = 128
NG = 64
NCLS = 18
NSC = 2
NT = 16
NW = NSC * NT
FPT = D // NW
N_PAD = 10240
EC = 4096
E_CHUNKS = 80
E_PAD = EC * E_CHUNKS
DEG_CHUNKS = E_PAD // NW // 128
ROW_BLK = 512
N_BLKS = N_PAD // ROW_BLK

_SC_MESH = plsc.VectorSubcoreMesh(core_axis_name="c", subcore_axis_name="s")
_SC_PARAMS = pltpu.CompilerParams(needs_layout_passes=False)


def _spmm_body(gt_hbm, src_hbm, dst_hbm, w_hbm, out,
               slab, accv, src0, src1, dst0, dst1, w0, w1, sem0, sem1):
    c = lax.axis_index("c")
    s = lax.axis_index("s")
    wid = c * NT + s
    srcb = (src0, src1)
    dstb = (dst0, dst1)
    wb = (w0, w1)
    sem = (sem0, sem1)

    pltpu.sync_copy(gt_hbm.at[pl.ds(wid * FPT, FPT)], slab)

    def az(i, carry):
        for f in range(FPT):
            accv[f, pl.ds(i * 16, 16)] = jnp.zeros((16,), f32)
        return carry
    lax.fori_loop(0, N_PAD // 16, az, 0)

    def fire(ch, b):
        es = pl.ds(ch * EC, EC)
        pltpu.async_copy(src_hbm.at[es], srcb[b], sem[b])
        pltpu.async_copy(dst_hbm.at[es], dstb[b], sem[b])
        pltpu.async_copy(w_hbm.at[es], wb[b], sem[b])

    def drain(ch, b):
        es = pl.ds(ch * EC, EC)
        pltpu.make_async_copy(src_hbm.at[es], srcb[b], sem[b]).wait()
        pltpu.make_async_copy(dst_hbm.at[es], dstb[b], sem[b]).wait()
        pltpu.make_async_copy(w_hbm.at[es], wb[b], sem[b]).wait()

    fire(0, 0)
    fire(1, 1)

    def pair(p, carry):
        for b in range(2):
            ch = p * 2 + b
            drain(ch, b)

            @plsc.parallel_loop(0, EC // 16, unroll=4)
            def grp(i):
                sl = pl.ds(i * 16, 16)
                s16 = srcb[b][sl]
                d16 = dstb[b][sl]
                w16 = wb[b][sl]
                for f in range(FPT):
                    ff = jnp.full((16,), f, i32)
                    vals = plsc.load_gather(slab, [ff, s16])
                    plsc.addupdate_scatter(accv, [ff, d16], vals * w16)
            nxt = ch + 2

            @pl.when(nxt < E_CHUNKS)
            def _():
                fire(nxt, b)
        return carry
    lax.fori_loop(0, E_CHUNKS // 2, pair, 0)

    pltpu.sync_copy(accv, out.at[pl.ds(wid * FPT, FPT)])


def _spmm(gt, srcf, dstf, wf):
    return pl.kernel(
        _spmm_body,
        out_type=jax.ShapeDtypeStruct((D, N_PAD), f32),
        mesh=_SC_MESH,
        compiler_params=_SC_PARAMS,
        scratch_types=[
            pltpu.VMEM((FPT, N_PAD), f32),
            pltpu.VMEM((FPT, N_PAD), f32),
            pltpu.VMEM((EC,), i32),
            pltpu.VMEM((EC,), i32),
            pltpu.VMEM((EC,), i32),
            pltpu.VMEM((EC,), i32),
            pltpu.VMEM((EC,), f32),
            pltpu.VMEM((EC,), f32),
            pltpu.SemaphoreType.DMA,
            pltpu.SemaphoreType.DMA,
        ],
    )(gt, srcf, dstf, wf)


def _deg_body(dst_hbm, w_hbm, out, degp, dst_v, w_v):
    c = lax.axis_index("c")
    s = lax.axis_index("s")
    wid = c * NT + s

    def dz(i, carry):
        degp[pl.ds(i * 16, 16)] = jnp.zeros((16,), f32)
        return carry
    lax.fori_loop(0, N_PAD // 16, dz, 0)

    pltpu.sync_copy(dst_hbm.at[wid], dst_v)
    pltpu.sync_copy(w_hbm.at[wid], w_v)

    @plsc.parallel_loop(0, DEG_CHUNKS, unroll=2)
    def eb(r):
        for gi in range(8):
            sl = pl.ds(gi * 16, 16)
            plsc.addupdate_scatter(degp, [dst_v[r, sl]], w_v[r, sl])

    pltpu.sync_copy(degp, out.at[wid])


def _deg(dst3, w3):
    return pl.kernel(
        _deg_body,
        out_type=jax.ShapeDtypeStruct((NW, N_PAD), f32),
        mesh=_SC_MESH,
        compiler_params=_SC_PARAMS,
        scratch_types=[
            pltpu.VMEM((N_PAD,), f32),
            pltpu.VMEM((DEG_CHUNKS, 128), i32),
            pltpu.VMEM((DEG_CHUNKS, 128), f32),
        ],
    )(dst3, w3)


def _dred_body(degp_ref, out_ref):
    out_ref[...] = lax.rsqrt(1.0 + jnp.sum(degp_ref[...], axis=0,
                                           keepdims=True))


def _dred(degp):
    return pl.pallas_call(
        _dred_body,
        grid=(N_BLKS,),
        in_specs=[pl.BlockSpec((NW, ROW_BLK), lambda i: (0, i))],
        out_specs=pl.BlockSpec((1, ROW_BLK), lambda i: (0, i)),
        out_shape=jax.ShapeDtypeStruct((1, N_PAD), f32),
    )(degp)


def _g0_body(x_ref, w0t_ref, dinv_ref, g0_ref):
    xw = lax.dot_general(w0t_ref[...], x_ref[...],
                         (((1,), (1,)), ((), ())),
                         preferred_element_type=f32)
    g0_ref[...] = dinv_ref[...] * xw


def _g0(x_pad, w0t, dinvT):
    return pl.pallas_call(
        _g0_body,
        grid=(N_BLKS,),
        in_specs=[
            pl.BlockSpec((ROW_BLK, D), lambda i: (i, 0)),
            pl.BlockSpec((D, D), lambda i: (0, 0)),
            pl.BlockSpec((1, ROW_BLK), lambda i: (0, i)),
        ],
        out_specs=pl.BlockSpec((D, ROW_BLK), lambda i: (0, i)),
        out_shape=jax.ShapeDtypeStruct((D, N_PAD), f32),
    )(x_pad, w0t, dinvT)


def _layer_body(p_ref, g_ref, dinv_ref, b_ref, wt_ref, out_ref):
    dv = dinv_ref[...]
    h = dv * (p_ref[...] + g_ref[...]) + b_ref[...]
    h = jnp.maximum(h, 0.0)
    out_ref[...] = dv * jnp.dot(wt_ref[...], h, preferred_element_type=f32)


def _layer(p, g, dinvT, bcol, wt):
    return pl.pallas_call(
        _layer_body,
        grid=(N_BLKS,),
        in_specs=[
            pl.BlockSpec((D, ROW_BLK), lambda i: (0, i)),
            pl.BlockSpec((D, ROW_BLK), lambda i: (0, i)),
            pl.BlockSpec((1, ROW_BLK), lambda i: (0, i)),
            pl.BlockSpec((D, 1), lambda i: (0, 0)),
            pl.BlockSpec((D, D), lambda i: (0, 0)),
        ],
        out_specs=pl.BlockSpec((D, ROW_BLK), lambda i: (0, i)),
        out_shape=jax.ShapeDtypeStruct((D, N_PAD), f32),
    )(p, g, dinvT, bcol, wt)


def _pool_body(p_ref, g_ref, dinv_ref, b_ref, batch_ref, out_ref):
    i = pl.program_id(0)
    h = dinv_ref[...] * (p_ref[...] + g_ref[...]) + b_ref[...]
    onehot = (batch_ref[...] ==
              lax.broadcasted_iota(i32, (ROW_BLK, NG), 1)).astype(f32)
    acc = jnp.dot(h, onehot, preferred_element_type=f32)

    @pl.when(i == 0)
    def _():
        out_ref[...] = acc

    @pl.when(i != 0)
    def _():
        out_ref[...] += acc


def _pool(p, g, dinvT, bcol, batch_col):
    return pl.pallas_call(
        _pool_body,
        grid=(N_BLKS,),
        in_specs=[
            pl.BlockSpec((D, ROW_BLK), lambda i: (0, i)),
            pl.BlockSpec((D, ROW_BLK), lambda i: (0, i)),
            pl.BlockSpec((1, ROW_BLK), lambda i: (0, i)),
            pl.BlockSpec((D, 1), lambda i: (0, 0)),
            pl.BlockSpec((ROW_BLK, 1), lambda i: (i, 0)),
        ],
        out_specs=pl.BlockSpec((D, NG), lambda i: (0, 0)),
        out_shape=jax.ShapeDtypeStruct((D, NG), f32),
    )(p, g, dinvT, bcol, batch_col)


def _head_body(pooled_ref, fc1wt_ref, fc1b_ref, fc2wt_ref, fc2b_ref,
               gam_ref, bet_ref, out_ref):
    p = pooled_ref[...]
    mean = jnp.mean(p, axis=1, keepdims=True)
    var = jnp.mean((p - mean) ** 2, axis=1, keepdims=True)
    hn = (p - mean) * lax.rsqrt(var + 1e-5) * gam_ref[...] + bet_ref[...]
    hf = jnp.dot(fc1wt_ref[...], hn, preferred_element_type=f32) + fc1b_ref[...]
    hf = jnp.maximum(hf, 0.0)
    logits = jnp.dot(fc2wt_ref[...], hf, preferred_element_type=f32) + fc2b_ref[...]
    row = lax.broadcasted_iota(i32, (D, NG), 0)
    lm = jnp.where(row < NCLS, logits, -1e30)
    mx = jnp.max(lm, axis=0, keepdims=True)
    lse = jnp.log(jnp.sum(jnp.exp(lm - mx), axis=0, keepdims=True)) + mx
    out_ref[...] = logits - lse


def _head(pooledT, fc1wt, fc1b_col, fc2wt_pad, fc2b_col, gam_col, bet_col):
    return pl.pallas_call(
        _head_body,
        out_shape=jax.ShapeDtypeStruct((D, NG), f32),
    )(pooledT, fc1wt, fc1b_col, fc2wt_pad, fc2b_col, gam_col, bet_col)


def kernel(x, edge_index, edge_type, batch, W, b,
           fc1_W, fc1_b, fc2_W, fc2_b, bn_gamma, bn_beta):
    n = x.shape[0]
    ne = edge_index.shape[1]
    n_conv = W.shape[0]

    src = edge_index[0].astype(i32)
    dst = edge_index[1].astype(i32)
    w = edge_type.astype(f32)

    epad = E_PAD - ne
    srcf = jnp.concatenate([src, jnp.zeros((epad,), i32)])
    dstf = jnp.concatenate([dst, jnp.zeros((epad,), i32)])
    wf = jnp.concatenate([w, jnp.zeros((epad,), f32)])
    dst3 = dstf.reshape(NW, DEG_CHUNKS, 128)
    w3 = wf.reshape(NW, DEG_CHUNKS, 128)

    x_pad = jnp.concatenate([x.astype(f32), jnp.zeros((N_PAD - n, D), f32)])
    batch_col = jnp.concatenate([batch.astype(i32),
                                 jnp.full((N_PAD - n,), NG, i32)]).reshape(N_PAD, 1)

    Wt = jnp.swapaxes(W, 1, 2).astype(f32)

    degp = _deg(dst3, w3)
    dinvT = _dred(degp)
    g = _g0(x_pad, Wt[0], dinvT)

    p = None
    for i in range(n_conv):
        p = _spmm(g, srcf, dstf, wf)
        if i + 1 < n_conv:
            g = _layer(p, g, dinvT, b[i].reshape(D, 1), Wt[i + 1])

    pooledT = _pool(p, g, dinvT, b[n_conv - 1].reshape(D, 1), batch_col)

    fc2wt_pad = jnp.zeros((D, D), f32).at[:NCLS, :].set(fc2_W.T.astype(f32))
    fc2b_col = jnp.zeros((D, 1), f32).at[:NCLS, 0].set(fc2_b.astype(f32))
    outT = _head(pooledT, fc1_W.T.astype(f32), fc1_b.reshape(D, 1).astype(f32),
                 fc2wt_pad, fc2b_col, bn_gamma.reshape(D, 1).astype(f32),
                 bn_beta.reshape(D, 1).astype(f32))
    return outT[:NCLS, :].T
